# baseline (speedup 1.0000x reference)
import functools

import jax
import jax.numpy as jnp
from jax import lax
from jax.experimental import pallas as pl
from jax.experimental.pallas import tpu as pltpu
from jax.experimental.pallas import tpu_sc as plsc

MAXLEN = 200
EMBED = 128
BATCH = 1024
ROWS = BATCH * MAXLEN
NC = 2
NS = 16
NW = NC * NS
SEQ_PW = BATCH // NW
HALF = MAXLEN // 2
WSPLIT = 96
LANES = 16
VPR = EMBED // LANES
NBUF = 3
UNROLL = 1


def _emb_kernel(idx_hbm, tok_hbm, pos_hbm, out_hbm, idx_v, pos_v,
                buf0, buf1, buf2, sg0, sg1, sg2, so0, so1, so2):
    wid = lax.axis_index("s") * NC + lax.axis_index("c")
    base = wid * SEQ_PW * MAXLEN

    pltpu.sync_copy(idx_hbm.at[wid], idx_v)

    bufs = [buf0, buf1, buf2]
    sg = [sg0, sg1, sg2]
    so = [so0, so1, so2]

    pos_d = pltpu.async_copy(pos_hbm, pos_v, so0)

    def add_rows(buf, lo, hi):
        @plsc.parallel_loop(lo, hi, step=UNROLL)
        def body(r):
            for j in range(VPR):
                sl = pl.ds(j * LANES, LANES)
                plsc.addupdate(buf.at[r, sl], pos_v[r, sl])

    gat_d = {}
    out_d = {}
    for t in range(SEQ_PW + 1):
        if t >= 1:
            c = t - 1
            b = c % NBUF
            if c == 0:
                pos_d.wait()
            for g in gat_d[c]:
                g.wait()
            buf = bufs[b]
            add_rows(buf, 0, MAXLEN)
            out_d[c] = (pltpu.async_copy(
                buf, out_hbm.at[pl.ds(base + c * MAXLEN, MAXLEN)], so[b]),)
        if t < SEQ_PW:
            b = t % NBUF
            if t >= NBUF:
                for o in out_d[t - NBUF]:
                    o.wait()
            gat_d[t] = tuple(
                pltpu.async_copy(tok_hbm.at[idx_v.at[t, h]],
                                 bufs[b].at[pl.ds(h * HALF, HALF)], sg[b])
                for h in range(2)
            )
    for c in range(SEQ_PW - NBUF, SEQ_PW):
        for o in out_d[c]:
            o.wait()


@functools.partial(jax.jit)
def _run(idx, tok, pos):
    mesh = plsc.VectorSubcoreMesh(core_axis_name="c", subcore_axis_name="s")
    f = functools.partial(
        pl.kernel,
        out_type=jax.ShapeDtypeStruct((ROWS, EMBED), jnp.float32),
        mesh=mesh,
        scratch_types=[
            pltpu.VMEM((SEQ_PW, 2, HALF), jnp.int32),
            pltpu.VMEM((MAXLEN, EMBED), jnp.float32),
            pltpu.VMEM((MAXLEN, EMBED), jnp.float32),
            pltpu.VMEM((MAXLEN, EMBED), jnp.float32),
            pltpu.VMEM((MAXLEN, EMBED), jnp.float32),
        ] + [pltpu.SemaphoreType.DMA] * 6,
    )(_emb_kernel)
    return f(idx, tok, pos)


def kernel(inputs, token_table, pos_table):
    idx = inputs.astype(jnp.int32).reshape(NW, SEQ_PW, 2, HALF)
    out = _run(idx, token_table, pos_table)
    return out.reshape(BATCH, MAXLEN, EMBED)


# speedup vs baseline: 1.5119x; 1.5119x over previous
import functools

import jax
import jax.numpy as jnp
from jax import lax
from jax.experimental import pallas as pl
from jax.experimental.pallas import tpu as pltpu
from jax.experimental.pallas import tpu_sc as plsc

MAXLEN = 200
EMBED = 128
BATCH = 1024
ROWS = BATCH * MAXLEN
NC = 2
NS = 16
NW = NC * NS
SEQ_PW = BATCH // NW
HALF = MAXLEN // 2
LANES = 16
VPR = EMBED // LANES
NBUF = 3
UNROLL = 1


def _emb_kernel(idx_hbm, tok_hbm, pos_hbm, out_hbm, idx_v, pos_v,
                buf0, buf1, buf2, sg0, sg1, sg2, so0, so1, so2):
    wid = lax.axis_index("s") * NC + lax.axis_index("c")
    base = wid * SEQ_PW * MAXLEN

    pltpu.sync_copy(idx_hbm.at[wid], idx_v)

    bufs = [buf0, buf1, buf2]
    sg = [sg0, sg1, sg2]
    so = [so0, so1, so2]

    pos_d = pltpu.async_copy(pos_hbm, pos_v, so0)

    def add_rows(buf, lo, hi):
        @plsc.parallel_loop(lo, hi, step=UNROLL)
        def body(r):
            for j in range(VPR):
                sl = pl.ds(j * LANES, LANES)
                plsc.addupdate(buf.at[r, sl], pos_v[r, sl])

    gat_d = {}
    out_d = {}
    for t in range(SEQ_PW + 1):
        if t < SEQ_PW:
            b = t % NBUF
            if t >= NBUF:
                for o in out_d[t - NBUF]:
                    o.wait()
            gat_d[t] = tuple(
                pltpu.async_copy(tok_hbm.at[idx_v.at[t, h]],
                                 bufs[b].at[pl.ds(h * HALF, HALF)], sg[b])
                for h in range(2)
            )
        if t >= 1:
            c = t - 1
            b = c % NBUF
            if c == 0:
                pos_d.wait()
            for g in gat_d[c]:
                g.wait()
            buf = bufs[b]
            add_rows(buf, 0, MAXLEN)
            out_d[c] = (pltpu.async_copy(
                buf, out_hbm.at[pl.ds(base + c * MAXLEN, MAXLEN)], so[b]),)
    for c in range(SEQ_PW - NBUF, SEQ_PW):
        for o in out_d[c]:
            o.wait()


@functools.partial(jax.jit)
def _run(idx, tok, pos):
    mesh = plsc.VectorSubcoreMesh(core_axis_name="c", subcore_axis_name="s")
    f = functools.partial(
        pl.kernel,
        out_type=jax.ShapeDtypeStruct((ROWS, EMBED), jnp.float32),
        mesh=mesh,
        scratch_types=[
            pltpu.VMEM((SEQ_PW, 2, HALF), jnp.int32),
            pltpu.VMEM((MAXLEN, EMBED), jnp.float32),
            pltpu.VMEM((MAXLEN, EMBED), jnp.float32),
            pltpu.VMEM((MAXLEN, EMBED), jnp.float32),
            pltpu.VMEM((MAXLEN, EMBED), jnp.float32),
        ] + [pltpu.SemaphoreType.DMA] * 6,
    )(_emb_kernel)
    return f(idx, tok, pos)


def kernel(inputs, token_table, pos_table):
    idx = inputs.astype(jnp.int32).reshape(NW, SEQ_PW, 2, HALF)
    out = _run(idx, token_table, pos_table)
    return out.reshape(BATCH, MAXLEN, EMBED)
